# baseline (speedup 1.0000x reference)
import functools

import jax
import jax.numpy as jnp
from jax import lax
from jax.experimental import pallas as pl
from jax.experimental.pallas import tpu as pltpu
from jax.experimental.pallas import tpu_sc as plsc

N, H, S, G, O = 100000, 256, 64, 512, 128
D = H + S

NC, NS = 2, 16
NW = NC * NS
L = 16
HL = H // L
CHUNK = 80
NCHUNK = N // CHUNK
CL = 38
CH = 40
NHI = (NCHUNK - NW * CL) // 2
FB = 16
TRASH = G
MAXF = 528
PW = 384


def _sc_pool(x, ids):
    mesh = plsc.VectorSubcoreMesh(core_axis_name="c", subcore_axis_name="s")

    @functools.partial(
        pl.kernel,
        out_type=[
            jax.ShapeDtypeStruct((NW, MAXF, PW), jnp.float32),
            jax.ShapeDtypeStruct((NW, MAXF, 128), jnp.float32),
        ],
        mesh=mesh,
        scratch_types=[
            pltpu.VMEM((CHUNK, H), jnp.float32),
            pltpu.VMEM((CHUNK, H), jnp.float32),
            pltpu.VMEM((CHUNK,), jnp.int32),
            pltpu.VMEM((CHUNK,), jnp.int32),
            pltpu.VMEM((FB, PW), jnp.float32),
            pltpu.VMEM((FB, 128), jnp.float32),
            pltpu.VMEM((FB, 128), jnp.float32),
            pltpu.VMEM((H,), jnp.float32),
            pltpu.SMEM((8,), jnp.int32),
            pltpu.SMEM((8,), jnp.float32),
            pltpu.SemaphoreType.DMA,
            pltpu.SemaphoreType.DMA,
        ],
        compiler_params=pltpu.CompilerParams(needs_layout_passes=False),
    )
    def pool(x_hbm, ids_hbm,
             parts_out, pids_out,
             rows0, rows1, ids0, ids1, flushv, fpid, tfbuf,
             acc_ref, smem_i, smem_f, sem0, sem1):
        c = lax.axis_index("c")
        s = lax.axis_index("s")
        wid = s * NC + c
        base = wid * CH - (CH - CL) * jnp.maximum(wid - NHI, 0)
        mychunks = jnp.where(wid < NHI, CH, CL)
        rows_b, ids_b, sems = (rows0, rows1), (ids0, ids1), (sem0, sem1)
        lane0 = lax.iota(jnp.int32, L) == 0
        zvec = jnp.zeros((L,), jnp.float32)
        trashf = zvec + jnp.float32(TRASH)

        def splat_i32(v):
            return jnp.zeros((L,), jnp.int32) + v

        def start_load(t, b):
            gc = base + t
            pltpu.async_copy(ids_hbm.at[pl.ds(gc * CHUNK, CHUNK)],
                             ids_b[b], sems[b])
            pltpu.async_copy(x_hbm.at[pl.ds(gc * CHUNK, CHUNK)],
                             rows_b[b], sems[b])

        def wait_load(b):
            pltpu.make_async_copy(ids_hbm.at[pl.ds(0, CHUNK)],
                                  ids_b[b], sems[b]).wait()
            pltpu.make_async_copy(x_hbm.at[pl.ds(0, CHUNK)],
                                  rows_b[b], sems[b]).wait()

        def reset_fpid():
            def _ri(i, cc):
                def _rm(m, cc2):
                    fpid[i, pl.ds(L * m, L)] = trashf
                    return cc2
                return lax.fori_loop(0, 128 // L, _rm, cc)
            lax.fori_loop(0, FB, _ri, 0)

        start_load(0, 0)
        start_load(1, 1)
        def _fill_tf(i, cc):
            def _fm(m, cc2):
                tfbuf[i, pl.ds(L * m, L)] = trashf
                return cc2
            return lax.fori_loop(0, 128 // L, _fm, cc)
        lax.fori_loop(0, FB, _fill_tf, 0)

        def _dd(dd, cc):
            pltpu.sync_copy(tfbuf, pids_out.at[wid, pl.ds(dd * FB, FB)])
            return cc
        lax.fori_loop(0, MAXF // FB, _dd, 0)

        def _zpad(i, cc):
            def _zm(m, cc2):
                flushv[i, pl.ds(L * m, L)] = zvec
                return cc2
            return lax.fori_loop(H // L, PW // L, _zm, cc)
        lax.fori_loop(0, FB, _zpad, 0)
        reset_fpid()

        def do_flush_now(curv_f):
            n = smem_i[0]
            rcf = smem_f[0]

            def _fm(m, cc):
                flushv[n, pl.ds(L * m, L)] = acc_ref[pl.ds(L * m, L)]
                return cc
            lax.fori_loop(0, HL, _fm, 0)
            flushv[n, pl.ds(H, L)] = jnp.where(lane0, rcf, 0.0)

            def _pm(m, cc):
                fpid[n, pl.ds(L * m, L)] = curv_f
                return cc
            lax.fori_loop(0, 128 // L, _pm, 0)
            smem_i[0] = n + 1

        def drain_now():
            d = smem_i[1]
            pltpu.sync_copy(flushv, parts_out.at[wid, pl.ds(d * FB, FB)])
            pltpu.sync_copy(fpid, pids_out.at[wid, pl.ds(d * FB, FB)])
            reset_fpid()
            smem_i[0] = 0
            smem_i[1] = d + 1

        def group_step(b, g):
            ids_g = ids_b[b][pl.ds(g * L, L)]
            curv = splat_i32(smem_i[2])
            same = jnp.all(ids_g == curv)

            @pl.when(same)
            def _fast():
                for m in range(HL):
                    t = [rows_b[b][g * L + k, pl.ds(L * m, L)]
                         for k in range(L)]
                    while len(t) > 1:
                        t = [u + v for u, v in zip(t[::2], t[1::2])]
                    acc_ref[pl.ds(L * m, L)] = (
                        acc_ref[pl.ds(L * m, L)] + t[0])
                smem_f[0] = smem_f[0] + jnp.float32(L)

            @pl.when(jnp.logical_not(same))
            def _slow():
                def _row(k, cc, b=b, g=g):
                    j = g * L + k
                    idv = plsc.load_gather(ids_b[b], [splat_i32(j)])
                    curv_k = splat_i32(smem_i[2])
                    eq = jnp.all(idv == curv_k)
                    rcf_k = smem_f[0]
                    flq = jnp.logical_and(jnp.logical_not(eq),
                                          rcf_k > 0.5)

                    @pl.when(flq)
                    def _fl(curv_k=curv_k):
                        do_flush_now(curv_k.astype(jnp.float32))

                    @pl.when(smem_i[0] == FB)
                    def _dr():
                        drain_now()

                    sel = jnp.where(eq, 1.0, 0.0)

                    def _am(m, cc2, j=j, sel=sel):
                        xv = rows_b[b][j, pl.ds(L * m, L)]
                        acc_ref[pl.ds(L * m, L)] = (
                            acc_ref[pl.ds(L * m, L)] * sel + xv)
                        return cc2
                    lax.fori_loop(0, HL, _am, 0)
                    smem_f[0] = jnp.where(eq, rcf_k + 1.0, 1.0)
                    smem_i[2] = jnp.max(idv)
                    return cc
                lax.fori_loop(0, L, _row, 0)

        def pair_body(tt, carry):
            for b in range(2):
                t = 2 * tt + b
                wait_load(b)

                def g_loop(g, cc, b=b):
                    group_step(b, g)
                    return cc
                lax.fori_loop(0, CHUNK // L, g_loop, 0)

                @pl.when(t + 2 < mychunks)
                def _(t=t, b=b):
                    start_load(t + 2, b)

            return carry

        smem_i[0] = 0
        smem_i[1] = 0
        smem_i[2] = 0
        smem_f[0] = 0.0
        for m in range(HL):
            acc_ref[pl.ds(L * m, L)] = zvec
        lax.fori_loop(0, mychunks // 2, pair_body, 0)

        @pl.when(smem_f[0] > 0.5)
        def _():
            do_flush_now(splat_i32(smem_i[2]).astype(jnp.float32))

        d = smem_i[1]
        pltpu.sync_copy(flushv, parts_out.at[wid, pl.ds(d * FB, FB)])
        pltpu.sync_copy(fpid, pids_out.at[wid, pl.ds(d * FB, FB)])

    return pool(x, ids)


def _elu(v):
    return jnp.where(v > 0.0, v, jnp.exp(jnp.minimum(v, 0.0)) - 1.0)


def _dot(a, b):
    return jnp.dot(a, b, preferred_element_type=jnp.float32,
                   precision=lax.Precision.HIGHEST)


def _mlp_body(parts_ref, pids_ref, st_ref, w1_ref, b1_ref,
              w2_ref, b2_ref, wo_ref, bo_ref, out_ref, acc_ref):
    w = pl.program_id(0)
    pidm = pids_ref[0]
    valid = pidm[:, 0:1] != jnp.float32(TRASH)
    pv = jnp.where(valid, parts_ref[0], 0.0)
    dn = (((0,), (0,)), ((), ()))
    for q in range(G // 128):
        iot = (lax.broadcasted_iota(jnp.int32, (MAXF, 128), 1)
               .astype(jnp.float32) + jnp.float32(128 * q))
        oh = (pidm == iot).astype(jnp.float32)
        contrib = lax.dot_general(oh, pv, dn,
                                  preferred_element_type=jnp.float32,
                                  precision=lax.Precision.HIGHEST)

        @pl.when(w == 0)
        def _(q=q, contrib=contrib):
            acc_ref[pl.ds(128 * q, 128), :] = contrib

        @pl.when(w > 0)
        def _(q=q, contrib=contrib):
            acc_ref[pl.ds(128 * q, 128), :] = (
                acc_ref[pl.ds(128 * q, 128), :] + contrib)

    @pl.when(w == NW - 1)
    def _():
        acc = acc_ref[...]
        pooled = acc[:, 0:H] / jnp.maximum(acc[:, H:H + 1], 1.0)
        h = (_dot(pooled, w1_ref[0:H, :])
             + _dot(st_ref[...], w1_ref[H:D, :]) + b1_ref[...])
        h = _elu(h)
        h = _elu(_dot(h, w2_ref[...]) + b2_ref[...])
        out_ref[...] = _dot(h, wo_ref[...]) + bo_ref[...]


def kernel(X, batch_ids, static_graph_features, W1, b1, W2, b2, Wout, bout):
    ids_flat = batch_ids.astype(jnp.int32)
    parts, pids = _sc_pool(X, ids_flat)
    zero = lambda w: (0, 0)
    return pl.pallas_call(
        _mlp_body,
        grid=(NW,),
        in_specs=[
            pl.BlockSpec((1, MAXF, PW), lambda w: (w, 0, 0)),
            pl.BlockSpec((1, MAXF, 128), lambda w: (w, 0, 0)),
            pl.BlockSpec((G, S), zero),
            pl.BlockSpec((D, D), zero),
            pl.BlockSpec((D,), lambda w: (0,)),
            pl.BlockSpec((D, D), zero),
            pl.BlockSpec((D,), lambda w: (0,)),
            pl.BlockSpec((D, O), zero),
            pl.BlockSpec((O,), lambda w: (0,)),
        ],
        out_specs=pl.BlockSpec((G, O), zero),
        scratch_shapes=[pltpu.VMEM((G, PW), jnp.float32)],
        out_shape=jax.ShapeDtypeStruct((G, O), jnp.float32),
    )(parts, pids, static_graph_features, W1, b1, W2, b2, Wout, bout)


# speedup vs baseline: 1.3663x; 1.3663x over previous
import functools

import jax
import jax.numpy as jnp
from jax import lax
from jax.experimental import pallas as pl
from jax.experimental.pallas import tpu as pltpu
from jax.experimental.pallas import tpu_sc as plsc

N, H, S, G, O = 100000, 256, 64, 512, 128
D = H + S

NC, NS = 2, 16
NW = NC * NS
CHUNK = 125
NCHUNK = N // CHUNK
CPW = NCHUNK // NW
IPAD = 128
TRASH = G
ACC_ROWS = 544
ZROWS = ACC_ROWS // NS
CNT_W = 16


def _sc_pool(x, ids_pad, zsum, zcnt, ones):
    mesh = plsc.VectorSubcoreMesh(core_axis_name="c", subcore_axis_name="s")

    @functools.partial(
        pl.kernel,
        out_type=[
            jax.ShapeDtypeStruct((NC, G, H), jnp.float32),
            jax.ShapeDtypeStruct((NC, G, CNT_W), jnp.float32),
        ],
        mesh=mesh,
        scratch_types=[
            pltpu.VMEM((IPAD, H), jnp.float32),
            pltpu.VMEM((IPAD, H), jnp.float32),
            pltpu.VMEM((IPAD,), jnp.int32),
            pltpu.VMEM((IPAD,), jnp.int32),
            pltpu.VMEM((IPAD, CNT_W), jnp.float32),
            pltpu.VMEM_SHARED((ACC_ROWS, H), jnp.float32),
            pltpu.VMEM_SHARED((ACC_ROWS, CNT_W), jnp.float32),
            pltpu.SemaphoreType.DMA,
            pltpu.SemaphoreType.DMA,
        ],
        compiler_params=pltpu.CompilerParams(use_tc_tiling_on_sc=False),
    )
    def pool(x_hbm, ids_hbm, zsum_hbm, zcnt_hbm, ones_hbm,
             sums_out, cnts_out, rows0, rows1, ids0, ids1, ones_v,
             acc_sh, cnt_sh, sem0, sem1):
        c = lax.axis_index("c")
        s = lax.axis_index("s")
        wid = s * NC + c
        base = wid * CPW
        rows_b, ids_b, sems = (rows0, rows1), (ids0, ids1), (sem0, sem1)

        def start_load(t, b):
            gc = base + t
            pltpu.async_copy(ids_hbm.at[gc], ids_b[b], sems[b])
            pltpu.async_copy(x_hbm.at[pl.ds(gc * CHUNK, CHUNK)],
                             rows_b[b].at[pl.ds(0, CHUNK)], sems[b])

        def wait_load(b):
            pltpu.make_async_copy(ids_hbm.at[0], ids_b[b], sems[b]).wait()
            pltpu.make_async_copy(x_hbm.at[pl.ds(0, CHUNK)],
                                  rows_b[b].at[pl.ds(0, CHUNK)],
                                  sems[b]).wait()

        start_load(0, 0)
        start_load(1, 1)
        pltpu.sync_copy(zsum_hbm.at[pl.ds(s * ZROWS, ZROWS)],
                        acc_sh.at[pl.ds(s * ZROWS, ZROWS)])
        pltpu.sync_copy(zcnt_hbm.at[pl.ds(s * ZROWS, ZROWS)],
                        cnt_sh.at[pl.ds(s * ZROWS, ZROWS)])
        pltpu.sync_copy(ones_hbm, ones_v)
        pltpu.sync_copy(zsum_hbm.at[pl.ds(0, IPAD - CHUNK)],
                        rows0.at[pl.ds(CHUNK, IPAD - CHUNK)])
        pltpu.sync_copy(zsum_hbm.at[pl.ds(0, IPAD - CHUNK)],
                        rows1.at[pl.ds(CHUNK, IPAD - CHUNK)])
        plsc.subcore_barrier()

        def body(tt, carry):
            for b in range(2):
                t = 2 * tt + b

                @pl.when(t < CPW)
                def _process(t=t, b=b):
                    wait_load(b)
                    pltpu.sync_copy(rows_b[b], acc_sh.at[ids_b[b]], add=True)
                    pltpu.sync_copy(ones_v, cnt_sh.at[ids_b[b]], add=True)

                    @pl.when(t + 2 < CPW)
                    def _prefetch(t=t, b=b):
                        start_load(t + 2, b)

            return carry

        lax.fori_loop(0, (CPW + 1) // 2, body, 0)
        plsc.subcore_barrier()

        @pl.when(s == 0)
        def _():
            pltpu.sync_copy(acc_sh.at[pl.ds(0, G)], sums_out.at[c])
            pltpu.sync_copy(cnt_sh.at[pl.ds(0, G)], cnts_out.at[c])

    return pool(x, ids_pad, zsum, zcnt, ones)


def _elu(v):
    return jnp.where(v > 0.0, v, jnp.exp(jnp.minimum(v, 0.0)) - 1.0)


def _dot(a, b):
    return jnp.dot(a, b, preferred_element_type=jnp.float32,
                   precision=lax.Precision.HIGHEST)


def _mlp_body(sums_ref, cnts_ref, st_ref, w1_ref, b1_ref, w2_ref, b2_ref,
              wo_ref, bo_ref, out_ref):
    sums = sums_ref[0] + sums_ref[1]
    cnt = cnts_ref[0, :, 0:1] + cnts_ref[1, :, 0:1]
    pooled = sums / jnp.maximum(cnt, 1.0)
    h = (_dot(pooled, w1_ref[0:H, :]) + _dot(st_ref[...], w1_ref[H:D, :])
         + b1_ref[...])
    h = _elu(h)
    h = _elu(_dot(h, w2_ref[...]) + b2_ref[...])
    out_ref[...] = _dot(h, wo_ref[...]) + bo_ref[...]


def kernel(X, batch_ids, static_graph_features, W1, b1, W2, b2, Wout, bout):
    ids = batch_ids.astype(jnp.int32).reshape(NCHUNK, CHUNK)
    ids_pad = jnp.full((NCHUNK, IPAD), TRASH, jnp.int32).at[:, :CHUNK].set(ids)
    zsum = jnp.zeros((ACC_ROWS, H), jnp.float32)
    zcnt = jnp.zeros((ACC_ROWS, CNT_W), jnp.float32)
    ones = jnp.ones((IPAD, CNT_W), jnp.float32)
    sums2, cnts2 = _sc_pool(X, ids_pad, zsum, zcnt, ones)
    return pl.pallas_call(
        _mlp_body,
        out_shape=jax.ShapeDtypeStruct((G, O), jnp.float32),
    )(sums2, cnts2, static_graph_features, W1, b1, W2, b2, Wout, bout)
